# baseline (speedup 1.0000x reference)
import math
from functools import partial

import jax
import jax.numpy as jnp
from jax.experimental import pallas as pl
from jax.experimental.pallas import tpu as pltpu


def _fused_kernel(a_ref, x_ref, w1_ref, b1_ref, w2_ref, b2_ref, pool_ref,
                  wc_ref, bc_ref, out_ref, h1_s, hg_s):
    p = pl.program_id(0)
    i = pl.program_id(1)
    n_rel, tile_m, _ = a_ref.shape

    @pl.when(p == 0)
    def _layer1():
        x = x_ref[...]
        acc = b1_ref[...]
        for r in range(n_rel):
            z = jnp.dot(a_ref[r], x, preferred_element_type=jnp.float32)
            acc = acc + jnp.dot(z, w1_ref[r],
                                preferred_element_type=jnp.float32)
        h1_s[pl.ds(i * tile_m, tile_m), :] = jnp.maximum(acc, 0.0)

    @pl.when(p == 1)
    def _layer2():
        h1 = h1_s[...]
        acc = b2_ref[...]
        for r in range(n_rel):
            z = jnp.dot(a_ref[r], h1, preferred_element_type=jnp.float32)
            acc = acc + jnp.dot(z, w2_ref[r],
                                preferred_element_type=jnp.float32)
        part = jnp.dot(pool_ref[0], acc, preferred_element_type=jnp.float32)

        @pl.when(i == 0)
        def _init():
            hg_s[...] = part

        @pl.when(i > 0)
        def _acc():
            hg_s[...] += part

    @pl.when(jnp.logical_and(p == 1, i == pl.num_programs(1) - 1))
    def _classifier():
        out_ref[...] = (jnp.dot(hg_s[...], wc_ref[...],
                                preferred_element_type=jnp.float32)
                        + bc_ref[0:1])


def _pad_to(a, shape):
    return jnp.pad(a, [(0, t - s) for s, t in zip(a.shape, shape)])


@partial(jax.jit, static_argnames=("tile_m",))
def _forward(a_norm, x, w1, b1_node, w2, b2_node, pool, wc, bc, *, tile_m=256):
    n_rel, n, _ = a_norm.shape
    f_in = x.shape[1]
    f_hid = w1.shape[2]
    n_graphs = pool.shape[0]
    n_classes = wc.shape[1]

    n_pad = tile_m * pl.cdiv(n, tile_m)
    g_pad = 8 * pl.cdiv(n_graphs, 8)
    n_tiles = n_pad // tile_m

    a_p = _pad_to(a_norm.astype(jnp.float32), (n_rel, n_pad, n_pad))
    x_p = _pad_to(x.astype(jnp.float32), (n_pad, f_in))
    w1_p = jnp.asarray(w1, jnp.float32)
    w2_p = jnp.asarray(w2, jnp.float32)
    b1_p = _pad_to(b1_node.astype(jnp.float32), (n_pad, f_hid))
    b2_p = _pad_to(b2_node.astype(jnp.float32), (n_pad, f_hid))
    pool_p = _pad_to(pool.astype(jnp.float32), (g_pad, n_pad))
    pool_tiles = pool_p.reshape(g_pad, n_tiles, tile_m).transpose(1, 0, 2)
    wc_p = jnp.asarray(wc, jnp.float32)
    bc_p = jnp.tile(jnp.asarray(bc, jnp.float32)[None, :], (8, 1))

    slab_bytes = n_rel * tile_m * n_pad * 4
    resident = (n_pad * (f_in + f_hid) * 4 + n_rel * f_hid * f_hid * 8
                + 4 * tile_m * f_hid * 4 + 4 * g_pad * (tile_m + f_hid) * 4)
    vmem_limit = int(min(2 * slab_bytes + resident + (8 << 20), 62 << 20))
    cparams = pltpu.CompilerParams(
        dimension_semantics=("arbitrary", "arbitrary"),
        vmem_limit_bytes=vmem_limit)

    out = pl.pallas_call(
        _fused_kernel,
        out_shape=jax.ShapeDtypeStruct((g_pad, wc.shape[1]), jnp.float32),
        grid=(2, n_tiles),
        in_specs=[
            pl.BlockSpec((n_rel, tile_m, n_pad), lambda p, i: (0, i, 0)),
            pl.BlockSpec((n_pad, f_in), lambda p, i: (0, 0)),
            pl.BlockSpec((n_rel, f_in, f_hid), lambda p, i: (0, 0, 0)),
            pl.BlockSpec((tile_m, f_hid), lambda p, i: (i, 0)),
            pl.BlockSpec((n_rel, f_hid, f_hid), lambda p, i: (0, 0, 0)),
            pl.BlockSpec((tile_m, f_hid), lambda p, i: (i, 0)),
            pl.BlockSpec((1, g_pad, tile_m), lambda p, i: (i, 0, 0)),
            pl.BlockSpec((f_hid, wc.shape[1]), lambda p, i: (0, 0)),
            pl.BlockSpec((8, wc.shape[1]), lambda p, i: (0, 0)),
        ],
        out_specs=pl.BlockSpec((g_pad, wc.shape[1]), lambda p, i: (0, 0)),
        scratch_shapes=[
            pltpu.VMEM((n_pad, f_hid), jnp.float32),
            pltpu.VMEM((g_pad, f_hid), jnp.float32),
        ],
        compiler_params=cparams,
    )(a_p, x_p, w1_p, b1_p, w2_p, b2_p, pool_tiles, wc_p, bc_p)

    return out[:n_graphs, :n_classes]


def kernel(a_norm, x, w1, b1_node, w2, b2_node, pool, wc, bc):
    return _forward(a_norm, x, w1, b1_node, w2, b2_node, pool, wc, bc,
                    tile_m=128)


# speedup vs baseline: 1.1373x; 1.1373x over previous
import math
from functools import partial

import jax
import jax.numpy as jnp
from jax.experimental import pallas as pl
from jax.experimental.pallas import tpu as pltpu


def _fused_kernel(a_ref, x_ref, w1_ref, b1_ref, w2_ref, b2_ref, pool_ref,
                  wc_ref, bc_ref, out_ref, h1_s, hg_s):
    p = pl.program_id(0)
    i = pl.program_id(1)
    n_rel, tile_m, _ = a_ref.shape

    @pl.when(p == 0)
    def _layer1():
        x = x_ref[...]
        acc = b1_ref[...]
        for r in range(n_rel):
            z = jnp.dot(a_ref[r], x, preferred_element_type=jnp.float32)
            acc = acc + jnp.dot(z, w1_ref[r],
                                preferred_element_type=jnp.float32)
        h1_s[pl.ds(i * tile_m, tile_m), :] = jnp.maximum(acc, 0.0)

    @pl.when(p == 1)
    def _layer2():
        h1 = h1_s[...]
        acc = b2_ref[...]
        for r in range(n_rel):
            z = jnp.dot(a_ref[r], h1, preferred_element_type=jnp.float32)
            acc = acc + jnp.dot(z, w2_ref[r],
                                preferred_element_type=jnp.float32)
        part = jnp.dot(pool_ref[0], acc, preferred_element_type=jnp.float32)

        @pl.when(i == 0)
        def _init():
            hg_s[...] = part

        @pl.when(i > 0)
        def _acc():
            hg_s[...] += part

    @pl.when(jnp.logical_and(p == 1, i == pl.num_programs(1) - 1))
    def _classifier():
        out_ref[...] = (jnp.dot(hg_s[...], wc_ref[...],
                                preferred_element_type=jnp.float32)
                        + bc_ref[0:1])


def _pad_to(a, shape):
    return jnp.pad(a, [(0, t - s) for s, t in zip(a.shape, shape)])


@partial(jax.jit, static_argnames=("tile_m",))
def _forward(a_norm, x, w1, b1_node, w2, b2_node, pool, wc, bc, *, tile_m=256):
    n_rel, n, _ = a_norm.shape
    f_in = x.shape[1]
    f_hid = w1.shape[2]
    n_graphs = pool.shape[0]
    n_classes = wc.shape[1]

    n_pad = tile_m * pl.cdiv(n, tile_m)
    g_pad = 8 * pl.cdiv(n_graphs, 8)
    n_tiles = n_pad // tile_m

    a_p = _pad_to(a_norm.astype(jnp.float32), (n_rel, n_pad, n_pad))
    x_p = _pad_to(x.astype(jnp.float32), (n_pad, f_in))
    w1_p = jnp.asarray(w1, jnp.float32)
    w2_p = jnp.asarray(w2, jnp.float32)
    b1_p = _pad_to(b1_node.astype(jnp.float32), (n_pad, f_hid))
    b2_p = _pad_to(b2_node.astype(jnp.float32), (n_pad, f_hid))
    pool_p = _pad_to(pool.astype(jnp.float32), (g_pad, n_pad))
    pool_tiles = pool_p.reshape(g_pad, n_tiles, tile_m).transpose(1, 0, 2)
    wc_p = jnp.asarray(wc, jnp.float32)
    bc_p = jnp.tile(jnp.asarray(bc, jnp.float32)[None, :], (8, 1))

    slab_bytes = n_rel * tile_m * n_pad * 4
    resident = (n_pad * (f_in + f_hid) * 4 + n_rel * f_hid * f_hid * 8
                + 4 * tile_m * f_hid * 4 + 4 * g_pad * (tile_m + f_hid) * 4)
    vmem_limit = int(min(2 * slab_bytes + resident + (8 << 20), 62 << 20))
    cparams = pltpu.CompilerParams(
        dimension_semantics=("arbitrary", "arbitrary"),
        vmem_limit_bytes=vmem_limit)

    out = pl.pallas_call(
        _fused_kernel,
        out_shape=jax.ShapeDtypeStruct((g_pad, wc.shape[1]), jnp.float32),
        grid=(2, n_tiles),
        in_specs=[
            pl.BlockSpec((n_rel, tile_m, n_pad), lambda p, i: (0, i, 0)),
            pl.BlockSpec((n_pad, f_in), lambda p, i: (0, 0)),
            pl.BlockSpec((n_rel, f_in, f_hid), lambda p, i: (0, 0, 0)),
            pl.BlockSpec((tile_m, f_hid), lambda p, i: (i, 0)),
            pl.BlockSpec((n_rel, f_hid, f_hid), lambda p, i: (0, 0, 0)),
            pl.BlockSpec((tile_m, f_hid), lambda p, i: (i, 0)),
            pl.BlockSpec((1, g_pad, tile_m), lambda p, i: (i, 0, 0)),
            pl.BlockSpec((f_hid, wc.shape[1]), lambda p, i: (0, 0)),
            pl.BlockSpec((8, wc.shape[1]), lambda p, i: (0, 0)),
        ],
        out_specs=pl.BlockSpec((g_pad, wc.shape[1]), lambda p, i: (0, 0)),
        scratch_shapes=[
            pltpu.VMEM((n_pad, f_hid), jnp.float32),
            pltpu.VMEM((g_pad, f_hid), jnp.float32),
        ],
        compiler_params=cparams,
    )(a_p, x_p, w1_p, b1_p, w2_p, b2_p, pool_tiles, wc_p, bc_p)

    return out[:n_graphs, :n_classes]


def kernel(a_norm, x, w1, b1_node, w2, b2_node, pool, wc, bc):
    return _forward(a_norm, x, w1, b1_node, w2, b2_node, pool, wc, bc,
                    tile_m=512)


# speedup vs baseline: 1.1685x; 1.0274x over previous
import math
from functools import partial

import jax
import jax.numpy as jnp
from jax.experimental import pallas as pl
from jax.experimental.pallas import tpu as pltpu


def _fused_kernel(a_ref, x_ref, w1_ref, b1_ref, w2_ref, b2_ref, pool_ref,
                  wc_ref, bc_ref, out_ref, h1_s, hg_s):
    p = pl.program_id(0)
    i = pl.program_id(1)
    n_rel, tile_m, _ = a_ref.shape

    @pl.when(p == 0)
    def _layer1():
        x = x_ref[...]
        acc = b1_ref[...]
        for r in range(n_rel):
            z = jnp.dot(a_ref[r], x, preferred_element_type=jnp.float32)
            acc = acc + jnp.dot(z, w1_ref[r],
                                preferred_element_type=jnp.float32)
        h1_s[pl.ds(i * tile_m, tile_m), :] = jnp.maximum(acc, 0.0)

    @pl.when(p == 1)
    def _layer2():
        h1 = h1_s[...]
        acc = b2_ref[...]
        for r in range(n_rel):
            z = jnp.dot(a_ref[r], h1, preferred_element_type=jnp.float32)
            acc = acc + jnp.dot(z, w2_ref[r],
                                preferred_element_type=jnp.float32)
        part = jnp.dot(pool_ref[...], acc, preferred_element_type=jnp.float32)

        @pl.when(i == 0)
        def _init():
            hg_s[...] = part

        @pl.when(i > 0)
        def _acc():
            hg_s[...] += part

    @pl.when(jnp.logical_and(p == 1, i == pl.num_programs(1) - 1))
    def _classifier():
        out_ref[...] = (jnp.dot(hg_s[...], wc_ref[...],
                                preferred_element_type=jnp.float32)
                        + bc_ref[0:1])


def _pad_to(a, shape):
    return jnp.pad(a, [(0, t - s) for s, t in zip(a.shape, shape)])


@partial(jax.jit, static_argnames=("tile_m",))
def _forward(a_norm, x, w1, b1_node, w2, b2_node, pool, wc, bc, *, tile_m=256):
    n_rel, n, _ = a_norm.shape
    f_in = x.shape[1]
    f_hid = w1.shape[2]
    n_graphs = pool.shape[0]
    n_classes = wc.shape[1]

    n_pad = tile_m * pl.cdiv(n, tile_m)
    g_pad = 8 * pl.cdiv(n_graphs, 8)
    n_tiles = n_pad // tile_m

    a_p = _pad_to(a_norm.astype(jnp.float32), (n_rel, n_pad, n_pad))
    x_p = _pad_to(x.astype(jnp.float32), (n_pad, f_in))
    w1_p = jnp.asarray(w1, jnp.float32)
    w2_p = jnp.asarray(w2, jnp.float32)
    b1_p = _pad_to(b1_node.astype(jnp.float32), (n_pad, f_hid))
    b2_p = _pad_to(b2_node.astype(jnp.float32), (n_pad, f_hid))
    pool_p = _pad_to(pool.astype(jnp.float32), (g_pad, n_pad))
    wc_p = jnp.asarray(wc, jnp.float32)
    bc_p = jnp.tile(jnp.asarray(bc, jnp.float32)[None, :], (8, 1))

    slab_bytes = n_rel * tile_m * n_pad * 4
    resident = (n_pad * (f_in + f_hid) * 4 + n_rel * f_hid * f_hid * 8
                + 4 * tile_m * f_hid * 4 + 4 * g_pad * (tile_m + f_hid) * 4)
    vmem_limit = int(min(2 * slab_bytes + resident + (8 << 20), 62 << 20))
    cparams = pltpu.CompilerParams(
        dimension_semantics=("arbitrary", "arbitrary"),
        vmem_limit_bytes=vmem_limit)

    out = pl.pallas_call(
        _fused_kernel,
        out_shape=jax.ShapeDtypeStruct((g_pad, wc.shape[1]), jnp.float32),
        grid=(2, n_tiles),
        in_specs=[
            pl.BlockSpec((n_rel, tile_m, n_pad), lambda p, i: (0, i, 0)),
            pl.BlockSpec((n_pad, f_in), lambda p, i: (0, 0)),
            pl.BlockSpec((n_rel, f_in, f_hid), lambda p, i: (0, 0, 0)),
            pl.BlockSpec((tile_m, f_hid), lambda p, i: (i, 0)),
            pl.BlockSpec((n_rel, f_hid, f_hid), lambda p, i: (0, 0, 0)),
            pl.BlockSpec((tile_m, f_hid), lambda p, i: (i, 0)),
            pl.BlockSpec((g_pad, tile_m), lambda p, i: (0, i)),
            pl.BlockSpec((f_hid, wc.shape[1]), lambda p, i: (0, 0)),
            pl.BlockSpec((8, wc.shape[1]), lambda p, i: (0, 0)),
        ],
        out_specs=pl.BlockSpec((g_pad, wc.shape[1]), lambda p, i: (0, 0)),
        scratch_shapes=[
            pltpu.VMEM((n_pad, f_hid), jnp.float32),
            pltpu.VMEM((g_pad, f_hid), jnp.float32),
        ],
        compiler_params=cparams,
    )(a_p, x_p, w1_p, b1_p, w2_p, b2_p, pool_p, wc_p, bc_p)

    return out[:n_graphs, :n_classes]


def kernel(a_norm, x, w1, b1_node, w2, b2_node, pool, wc, bc):
    return _forward(a_norm, x, w1, b1_node, w2, b2_node, pool, wc, bc,
                    tile_m=256)


# speedup vs baseline: 1.2111x; 1.0365x over previous
import math
from functools import partial

import jax
import jax.numpy as jnp
from jax.experimental import pallas as pl
from jax.experimental.pallas import tpu as pltpu


def _fused_kernel(a_ref, x_ref, w1_ref, b1_ref, w2_ref, b2_ref, pool_ref,
                  wc_ref, bc_ref, out_ref, h1_s, hg_s):
    p = pl.program_id(0)
    i = pl.program_id(1)
    n_rel, tile_m, _ = a_ref.shape

    @pl.when(p == 0)
    def _layer1():
        x = x_ref[...]
        acc = b1_ref[...]
        for r in range(n_rel):
            z = jnp.dot(a_ref[r], x, preferred_element_type=jnp.float32)
            acc = acc + jnp.dot(z, w1_ref[r],
                                preferred_element_type=jnp.float32)
        h1_s[pl.ds(i * tile_m, tile_m), :] = jnp.maximum(acc, 0.0)

    @pl.when(p == 1)
    def _layer2():
        h1 = h1_s[...]
        acc = b2_ref[...]
        for r in range(n_rel):
            z = jnp.dot(a_ref[r], h1, preferred_element_type=jnp.float32)
            acc = acc + jnp.dot(z, w2_ref[r],
                                preferred_element_type=jnp.float32)
        part = jnp.dot(pool_ref[...], acc, preferred_element_type=jnp.float32)

        @pl.when(i == 0)
        def _init():
            hg_s[...] = part

        @pl.when(i > 0)
        def _acc():
            hg_s[...] += part

    @pl.when(jnp.logical_and(p == 1, i == pl.num_programs(1) - 1))
    def _classifier():
        out_ref[...] = (jnp.dot(hg_s[...], wc_ref[...],
                                preferred_element_type=jnp.float32)
                        + bc_ref[0:1])


def _pad_to(a, shape):
    return jnp.pad(a, [(0, t - s) for s, t in zip(a.shape, shape)])


@partial(jax.jit, static_argnames=("tile_m",))
def _forward(a_norm, x, w1, b1_node, w2, b2_node, pool, wc, bc, *, tile_m=256):
    n_rel, n, _ = a_norm.shape
    f_in = x.shape[1]
    f_hid = w1.shape[2]
    n_graphs = pool.shape[0]
    n_classes = wc.shape[1]

    n_pad = tile_m * pl.cdiv(n, tile_m)
    g_pad = 8 * pl.cdiv(n_graphs, 8)
    n_tiles = n_pad // tile_m

    a_p = _pad_to(a_norm.astype(jnp.float32), (n_rel, n_pad, n_pad))
    x_p = _pad_to(x.astype(jnp.float32), (n_pad, f_in))
    w1_p = jnp.asarray(w1, jnp.float32)
    w2_p = jnp.asarray(w2, jnp.float32)
    b1_p = _pad_to(b1_node.astype(jnp.float32), (n_pad, f_hid))
    b2_p = _pad_to(b2_node.astype(jnp.float32), (n_pad, f_hid))
    pool_p = _pad_to(pool.astype(jnp.float32), (g_pad, n_pad))
    wc_p = jnp.asarray(wc, jnp.float32)
    bc_p = jnp.tile(jnp.asarray(bc, jnp.float32)[None, :], (8, 1))

    slab_bytes = n_rel * tile_m * n_pad * 4
    resident = (n_pad * (f_in + f_hid) * 4 + n_rel * f_hid * f_hid * 8
                + 4 * tile_m * f_hid * 4 + 4 * g_pad * (tile_m + f_hid) * 4)
    vmem_limit = int(min(2 * slab_bytes + resident + (8 << 20), 62 << 20))
    cparams = pltpu.CompilerParams(
        dimension_semantics=("arbitrary", "arbitrary"),
        vmem_limit_bytes=vmem_limit)

    out = pl.pallas_call(
        _fused_kernel,
        out_shape=jax.ShapeDtypeStruct((g_pad, wc.shape[1]), jnp.float32),
        grid=(2, n_tiles),
        in_specs=[
            pl.BlockSpec((n_rel, tile_m, n_pad), lambda p, i: (0, i, 0)),
            pl.BlockSpec((n_pad, f_in), lambda p, i: (0, 0)),
            pl.BlockSpec((n_rel, f_in, f_hid), lambda p, i: (0, 0, 0)),
            pl.BlockSpec((tile_m, f_hid), lambda p, i: (i * (1 - p), 0)),
            pl.BlockSpec((n_rel, f_hid, f_hid), lambda p, i: (0, 0, 0)),
            pl.BlockSpec((tile_m, f_hid), lambda p, i: (i * p, 0)),
            pl.BlockSpec((g_pad, tile_m), lambda p, i: (0, i * p)),
            pl.BlockSpec((f_hid, wc.shape[1]), lambda p, i: (0, 0)),
            pl.BlockSpec((8, wc.shape[1]), lambda p, i: (0, 0)),
        ],
        out_specs=pl.BlockSpec((g_pad, wc.shape[1]), lambda p, i: (0, 0)),
        scratch_shapes=[
            pltpu.VMEM((n_pad, f_hid), jnp.float32),
            pltpu.VMEM((g_pad, f_hid), jnp.float32),
        ],
        compiler_params=cparams,
    )(a_p, x_p, w1_p, b1_p, w2_p, b2_p, pool_p, wc_p, bc_p)

    return out[:n_graphs, :n_classes]


def kernel(a_norm, x, w1, b1_node, w2, b2_node, pool, wc, bc):
    return _forward(a_norm, x, w1, b1_node, w2, b2_node, pool, wc, bc,
                    tile_m=256)
